# baseline (speedup 1.0000x reference)
import jax
import jax.numpy as jnp
from jax import lax
from jax.experimental import pallas as pl
from jax.experimental.pallas import tpu as pltpu
from jax.experimental.pallas import tpu_sc as plsc

_NUM_CLASSES = 1000000
_FEAT_DIM = 64
_BATCH = 16384
_LAMBDA_C = 1.0

_NC = 2
_NS = 16
_NW = _NC * _NS
_ROWS_W = _BATCH // _NW
_LANES = 16
_KCHUNK = 32
_NKC = _ROWS_W // _KCHUNK
_BLK = _KCHUNK * _LANES


def _cl_body(featT_hbm, lab_hbm, centT_hbm, out_hbm,
             lab_v, feat_v, gath_a, gath_b, acc_v,
             semf, sem0, sem1):
    wid = lax.axis_index("s") * _NC + lax.axis_index("c")
    base = wid * _ROWS_W
    gbufs = [gath_a, gath_b]
    sems = [sem0, sem1]
    iota = lax.iota(jnp.int32, _LANES)

    pltpu.sync_copy(lab_hbm.at[pl.ds(base, _ROWS_W)], lab_v)
    fcp = pltpu.async_copy(featT_hbm.at[:, pl.ds(base, _ROWS_W)], feat_v,
                           semf)

    def issue_chunk(c):
        gbuf = gbufs[c % 2]

        def issue(g, carry, c=c, gbuf=gbuf):
            vec = lab_v[pl.ds(c * _KCHUNK + g * _LANES, _LANES)]
            for l in range(_LANES):
                r16 = pl.multiple_of((vec[l] >> 4) << 4, _LANES)
                pltpu.async_copy(
                    centT_hbm.at[:, pl.ds(r16, _LANES)],
                    gbuf.at[:, pl.ds((g * _LANES + l) * _LANES, _LANES)],
                    sems[c % 2])
            return carry
        lax.fori_loop(0, _KCHUNK // _LANES, issue, 0)

    issue_chunk(0)
    issue_chunk(1)
    fcp.wait()

    acc = jnp.zeros((_LANES,), jnp.float32)
    for c in range(_NKC):
        pltpu.make_async_copy(centT_hbm.at[:, pl.ds(0, _BLK)],
                              gbufs[c % 2], sems[c % 2]).wait()
        gbuf = gbufs[c % 2]

        for g in range(_KCHUNK // _LANES):
            vec = lab_v[pl.ds(c * _KCHUNK + g * _LANES, _LANES)]
            idxv = (g * _LANES + iota) * _LANES + (vec & (_LANES - 1))
            fbase = c * _KCHUNK + g * _LANES

            def dim_step(d, acc, gbuf=gbuf, idxv=idxv, fbase=fbase):
                f = feat_v[d, pl.ds(fbase, _LANES)]
                ce = plsc.load_gather(
                    gbuf, [jnp.full((_LANES,), d, jnp.int32), idxv])
                dd = f - ce
                return acc + dd * dd

            acc = lax.fori_loop(0, _FEAT_DIM, dim_step, acc)

        if c + 2 < _NKC:
            issue_chunk(c + 2)

    acc_v[...] = acc * (_LAMBDA_C / float(_BATCH * _FEAT_DIM))
    pltpu.sync_copy(acc_v, out_hbm.at[wid])


@jax.jit
def kernel(features, labels, centers):
    mesh = plsc.VectorSubcoreMesh(core_axis_name="c", subcore_axis_name="s")
    partials = pl.kernel(
        _cl_body,
        mesh=mesh,
        compiler_params=pltpu.CompilerParams(needs_layout_passes=False,
                                             use_tc_tiling_on_sc=False),
        out_type=jax.ShapeDtypeStruct((_NW, _LANES), jnp.float32),
        scratch_types=[
            pltpu.VMEM((_ROWS_W,), jnp.int32),
            pltpu.VMEM((_FEAT_DIM, _ROWS_W), jnp.float32),
            pltpu.VMEM((_FEAT_DIM, _BLK), jnp.float32),
            pltpu.VMEM((_FEAT_DIM, _BLK), jnp.float32),
            pltpu.VMEM((_LANES,), jnp.float32),
            pltpu.SemaphoreType.DMA,
            pltpu.SemaphoreType.DMA,
            pltpu.SemaphoreType.DMA,
        ],
    )(features.T, labels.astype(jnp.int32), centers.T)
    return jnp.sum(partials)


# speedup vs baseline: 14.0527x; 14.0527x over previous
import jax
import jax.numpy as jnp
from jax import lax
from jax.experimental import pallas as pl
from jax.experimental.pallas import tpu as pltpu
from jax.experimental.pallas import tpu_sc as plsc

_NUM_CLASSES = 1000000
_FEAT_DIM = 64
_BATCH = 16384
_LAMBDA_C = 1.0

_NC = 2
_NS = 16
_NW = _NC * _NS
_ROWS_W = _BATCH // _NW
_CHUNK = 128
_NCHUNK = _ROWS_W // _CHUNK
_LANES = 16
_GROUPS = _FEAT_DIM // _LANES


def _cl_body(feat_hbm, lab_hbm, cent_hbm, out_hbm,
             lab_v, feat_a, feat_b, rows_v, acc_v,
             semf, sem0, sem1, sem2, sem3):
    wid = lax.axis_index("s") * _NC + lax.axis_index("c")
    base = wid * _ROWS_W
    row_sems = [sem0, sem1, sem2, sem3]
    fbufs = [feat_a, feat_b]

    pltpu.sync_copy(lab_hbm.at[pl.ds(base, _ROWS_W)], lab_v)

    fcps = [pltpu.async_copy(feat_hbm.at[pl.ds(base, _CHUNK), :],
                             feat_a, semf)]

    for c in range(_NCHUNK):
        def issue(g, carry, c=c):
            vec = lab_v[pl.ds(c * _CHUNK + g * _LANES, _LANES)]
            for l in range(_LANES):
                r = vec[l]
                pltpu.async_copy(
                    cent_hbm.at[pl.ds(r, 1), :],
                    rows_v.at[pl.ds(c * _CHUNK + g * _LANES + l, 1), :],
                    row_sems[c])
            return carry
        lax.fori_loop(0, _CHUNK // _LANES, issue, 0)

    acc = jnp.zeros((_LANES,), jnp.float32)
    for c in range(_NCHUNK):
        if c + 1 < _NCHUNK:
            fcps.append(
                pltpu.async_copy(
                    feat_hbm.at[pl.ds(base + (c + 1) * _CHUNK, _CHUNK), :],
                    fbufs[(c + 1) % 2], semf))
        fcps[c].wait()
        pltpu.make_async_copy(cent_hbm.at[pl.ds(0, _CHUNK), :],
                              rows_v.at[pl.ds(c * _CHUNK, _CHUNK), :],
                              row_sems[c]).wait()

        fbuf = fbufs[c % 2]

        def row_step(i, acc, c=c, fbuf=fbuf):
            for j in range(_GROUPS):
                f = fbuf[i, pl.ds(j * _LANES, _LANES)]
                ce = rows_v[c * _CHUNK + i, pl.ds(j * _LANES, _LANES)]
                d = f - ce
                acc = acc + d * d
            return acc

        acc = lax.fori_loop(0, _CHUNK, row_step, acc)

    acc_v[...] = acc * (_LAMBDA_C / float(_BATCH * _FEAT_DIM))
    pltpu.sync_copy(acc_v, out_hbm.at[wid])


@jax.jit
def kernel(features, labels, centers):
    mesh = plsc.VectorSubcoreMesh(core_axis_name="c", subcore_axis_name="s")
    partials = pl.kernel(
        _cl_body,
        mesh=mesh,
        out_type=jax.ShapeDtypeStruct((_NW, _LANES), jnp.float32),
        scratch_types=[
            pltpu.VMEM((_ROWS_W,), jnp.int32),
            pltpu.VMEM((_CHUNK, _FEAT_DIM), jnp.float32),
            pltpu.VMEM((_CHUNK, _FEAT_DIM), jnp.float32),
            pltpu.VMEM((_ROWS_W, _FEAT_DIM), jnp.float32),
            pltpu.VMEM((_LANES,), jnp.float32),
            pltpu.SemaphoreType.DMA,
            pltpu.SemaphoreType.DMA,
            pltpu.SemaphoreType.DMA,
            pltpu.SemaphoreType.DMA,
            pltpu.SemaphoreType.DMA,
        ],
    )(features, labels.astype(jnp.int32), centers)
    return jnp.sum(partials)
